# baseline (speedup 1.0000x reference)
import jax
import jax.numpy as jnp
from jax.experimental import pallas as pl
from jax.experimental.pallas import tpu as pltpu

_RA = 200
_RB = 1000


def _xw_kernel(x_ref, w_ref, o_ref):
    o_ref[...] = jnp.dot(
        x_ref[...], w_ref[...], preferred_element_type=jnp.float32
    ).astype(jnp.bfloat16)


def _pass_a_kernel(alpha_ref, beta_ref, adj_ref, xw_ref, b1_ref,
                   h0_ref, h0b_ref, den_ref, adjb_ref, adjf8_ref):
    a = adj_ref[...]
    deg = jnp.sum(a, axis=1, keepdims=True)
    den_ref[...] = alpha_ref[0, 0] + beta_ref[0, 0] * deg
    ab = a.astype(jnp.bfloat16)
    adjb_ref[...] = ab
    adjf8_ref[...] = a.astype(jnp.float8_e4m3fn)
    acc = jnp.dot(ab, xw_ref[...], preferred_element_type=jnp.float32)
    h0 = jnp.maximum(acc + b1_ref[...], 0.0)
    h0_ref[...] = h0
    h0b_ref[...] = h0.astype(jnp.bfloat16)


def _f8_dot(a_f8, b):
    f8 = jnp.float8_e4m3fn
    hi = b.astype(f8)
    lo = (b - hi.astype(jnp.bfloat16)).astype(f8)
    return (jnp.dot(a_f8, hi, preferred_element_type=jnp.float32)
            + jnp.dot(a_f8, lo, preferred_element_type=jnp.float32))


def _crf_kernel(alpha_ref, beta_ref, adjb_ref, htb_ref, h0_ref, den_ref,
                out_ref):
    dot = _f8_dot(adjb_ref[...], htb_ref[...])
    ht = (alpha_ref[0, 0] * h0_ref[...] + beta_ref[0, 0] * dot) / den_ref[...]
    out_ref[...] = ht.astype(jnp.bfloat16)


def _crf_ln_kernel(alpha_ref, beta_ref, adjb_ref, htb_ref, h0_ref, den_ref,
                   g_ref, lb_ref, w2_ref, q_ref):
    dot = _f8_dot(adjb_ref[...], htb_ref[...])
    h = (alpha_ref[0, 0] * h0_ref[...] + beta_ref[0, 0] * dot) / den_ref[...]
    mu = jnp.mean(h, axis=1, keepdims=True)
    var = jnp.mean((h - mu) * (h - mu), axis=1, keepdims=True)
    hn = (h - mu) * jax.lax.rsqrt(var + 1e-5) * g_ref[...] + lb_ref[...]
    q = jnp.dot(hn, w2_ref[...], preferred_element_type=jnp.float32)
    q_ref[...] = q.astype(jnp.bfloat16)


def _gc2_kernel(adjb_ref, qb_ref, b2_ref, out_ref):
    logits = jnp.dot(adjb_ref[...], qb_ref[...],
                     preferred_element_type=jnp.float32) + b2_ref[...]
    m = jnp.max(logits, axis=1, keepdims=True)
    lse = jnp.log(jnp.sum(jnp.exp(logits - m), axis=1, keepdims=True)) + m
    out_ref[...] = logits - lse


def kernel(x, adj, W1, b1, W2, b2, ln_gamma, ln_beta, crf_alpha, crf_beta):
    n, nfeat = x.shape
    nhid = W1.shape[1]
    ncls = W2.shape[1]
    assert n % _RA == 0 and n % _RB == 0, (n, _RA, _RB)
    nblk_a = n // _RA
    nblk_b = n // _RB
    f32 = jnp.float32
    bf16 = jnp.bfloat16

    alpha = jnp.reshape(crf_alpha.astype(f32), (1, 1))
    beta = jnp.reshape(crf_beta.astype(f32), (1, 1))
    b1r = jnp.reshape(b1.astype(f32), (1, nhid))
    b2r = jnp.reshape(b2.astype(f32), (1, ncls))
    gr = jnp.reshape(ln_gamma.astype(f32), (1, nhid))
    lbr = jnp.reshape(ln_beta.astype(f32), (1, nhid))
    w2 = W2.astype(f32)

    xwb = pl.pallas_call(
        _xw_kernel,
        out_shape=jax.ShapeDtypeStruct((n, nhid), bf16),
    )(x, W1)

    params = pltpu.CompilerParams(dimension_semantics=("arbitrary",))
    row_blk_a = pl.BlockSpec((_RA, n), lambda i: (i, 0))
    hid_blk_a = pl.BlockSpec((_RA, nhid), lambda i: (i, 0))
    one_blk_a = pl.BlockSpec((_RA, 1), lambda i: (i, 0))
    row_blk_b = pl.BlockSpec((_RB, n), lambda i: (i, 0))
    hid_blk_b = pl.BlockSpec((_RB, nhid), lambda i: (i, 0))
    one_blk_b = pl.BlockSpec((_RB, 1), lambda i: (i, 0))
    scalar = pl.BlockSpec((1, 1), lambda i: (0, 0))

    f8 = jnp.float8_e4m3fn
    h0, h0b, den, adjb, adjf8 = pl.pallas_call(
        _pass_a_kernel,
        grid=(nblk_a,),
        in_specs=[scalar, scalar, row_blk_a,
                  pl.BlockSpec((n, nhid), lambda i: (0, 0)),
                  pl.BlockSpec((1, nhid), lambda i: (0, 0))],
        out_specs=[hid_blk_a, hid_blk_a, one_blk_a, row_blk_a, row_blk_a],
        out_shape=[jax.ShapeDtypeStruct((n, nhid), f32),
                   jax.ShapeDtypeStruct((n, nhid), bf16),
                   jax.ShapeDtypeStruct((n, 1), f32),
                   jax.ShapeDtypeStruct((n, n), bf16),
                   jax.ShapeDtypeStruct((n, n), f8)],
        compiler_params=params,
    )(alpha, beta, adj, xwb, b1r)

    crf_call = pl.pallas_call(
        _crf_kernel,
        grid=(nblk_b,),
        in_specs=[scalar, scalar, row_blk_b,
                  pl.BlockSpec((n, nhid), lambda i: (0, 0)),
                  hid_blk_b, one_blk_b],
        out_specs=hid_blk_b,
        out_shape=jax.ShapeDtypeStruct((n, nhid), bf16),
        compiler_params=params,
    )
    ht = h0b
    for _ in range(2):
        ht = crf_call(alpha, beta, adjf8, ht, h0, den)

    qb = pl.pallas_call(
        _crf_ln_kernel,
        grid=(nblk_b,),
        in_specs=[scalar, scalar, row_blk_b,
                  pl.BlockSpec((n, nhid), lambda i: (0, 0)),
                  hid_blk_b, one_blk_b,
                  pl.BlockSpec((1, nhid), lambda i: (0, 0)),
                  pl.BlockSpec((1, nhid), lambda i: (0, 0)),
                  pl.BlockSpec((nhid, ncls), lambda i: (0, 0))],
        out_specs=pl.BlockSpec((_RB, ncls), lambda i: (i, 0)),
        out_shape=jax.ShapeDtypeStruct((n, ncls), bf16),
        compiler_params=params,
    )(alpha, beta, adjf8, ht, h0, den, gr, lbr, w2)

    out = pl.pallas_call(
        _gc2_kernel,
        grid=(nblk_b,),
        in_specs=[row_blk_b,
                  pl.BlockSpec((n, ncls), lambda i: (0, 0)),
                  pl.BlockSpec((1, ncls), lambda i: (0, 0))],
        out_specs=pl.BlockSpec((_RB, ncls), lambda i: (i, 0)),
        out_shape=jax.ShapeDtypeStruct((n, ncls), f32),
        compiler_params=params,
    )(adjb, qb, b2r)
    return out


# speedup vs baseline: 1.0209x; 1.0209x over previous
import jax
import jax.numpy as jnp
from jax.experimental import pallas as pl
from jax.experimental.pallas import tpu as pltpu

_RA = 200
_RB = 1000


def _xw_kernel(x_ref, w_ref, o_ref):
    o_ref[...] = jnp.dot(
        x_ref[...], w_ref[...], preferred_element_type=jnp.float32
    ).astype(jnp.bfloat16)


def _pass_a_kernel(alpha_ref, beta_ref, adj_ref, xw_ref, b1_ref,
                   h0_ref, h0b_ref, den_ref, adjb_ref, adjf8_ref):
    a = adj_ref[...]
    deg = jnp.sum(a, axis=1, keepdims=True)
    den_ref[...] = alpha_ref[0, 0] + beta_ref[0, 0] * deg
    ab = a.astype(jnp.bfloat16)
    adjb_ref[...] = ab
    adjf8_ref[...] = a.astype(jnp.float8_e4m3fn)
    acc = jnp.dot(ab, xw_ref[...], preferred_element_type=jnp.float32)
    h0 = jnp.maximum(acc + b1_ref[...], 0.0)
    h0_ref[...] = h0
    h0b_ref[...] = h0.astype(jnp.bfloat16)


def _crf_kernel(alpha_ref, beta_ref, adjb_ref, htb_ref, h0_ref, den_ref,
                out_ref):
    dot = jnp.dot(adjb_ref[...].astype(jnp.bfloat16), htb_ref[...],
                  preferred_element_type=jnp.float32)
    ht = (alpha_ref[0, 0] * h0_ref[...] + beta_ref[0, 0] * dot) / den_ref[...]
    out_ref[...] = ht.astype(jnp.bfloat16)


def _crf_ln_kernel(alpha_ref, beta_ref, adjb_ref, htb_ref, h0_ref, den_ref,
                   g_ref, lb_ref, w2_ref, q_ref):
    dot = jnp.dot(adjb_ref[...].astype(jnp.bfloat16), htb_ref[...],
                  preferred_element_type=jnp.float32)
    h = (alpha_ref[0, 0] * h0_ref[...] + beta_ref[0, 0] * dot) / den_ref[...]
    mu = jnp.mean(h, axis=1, keepdims=True)
    var = jnp.mean((h - mu) * (h - mu), axis=1, keepdims=True)
    hn = (h - mu) * jax.lax.rsqrt(var + 1e-5) * g_ref[...] + lb_ref[...]
    q = jnp.dot(hn, w2_ref[...], preferred_element_type=jnp.float32)
    q_ref[...] = q.astype(jnp.bfloat16)


def _gc2_kernel(adjb_ref, qb_ref, b2_ref, out_ref):
    logits = jnp.dot(adjb_ref[...], qb_ref[...],
                     preferred_element_type=jnp.float32) + b2_ref[...]
    m = jnp.max(logits, axis=1, keepdims=True)
    lse = jnp.log(jnp.sum(jnp.exp(logits - m), axis=1, keepdims=True)) + m
    out_ref[...] = logits - lse


def kernel(x, adj, W1, b1, W2, b2, ln_gamma, ln_beta, crf_alpha, crf_beta):
    n, nfeat = x.shape
    nhid = W1.shape[1]
    ncls = W2.shape[1]
    assert n % _RA == 0 and n % _RB == 0, (n, _RA, _RB)
    nblk_a = n // _RA
    nblk_b = n // _RB
    f32 = jnp.float32
    bf16 = jnp.bfloat16

    alpha = jnp.reshape(crf_alpha.astype(f32), (1, 1))
    beta = jnp.reshape(crf_beta.astype(f32), (1, 1))
    b1r = jnp.reshape(b1.astype(f32), (1, nhid))
    b2r = jnp.reshape(b2.astype(f32), (1, ncls))
    gr = jnp.reshape(ln_gamma.astype(f32), (1, nhid))
    lbr = jnp.reshape(ln_beta.astype(f32), (1, nhid))
    w2 = W2.astype(f32)

    xwb = pl.pallas_call(
        _xw_kernel,
        out_shape=jax.ShapeDtypeStruct((n, nhid), bf16),
    )(x, W1)

    params = pltpu.CompilerParams(dimension_semantics=("arbitrary",))
    row_blk_a = pl.BlockSpec((_RA, n), lambda i: (i, 0))
    hid_blk_a = pl.BlockSpec((_RA, nhid), lambda i: (i, 0))
    one_blk_a = pl.BlockSpec((_RA, 1), lambda i: (i, 0))
    row_blk_b = pl.BlockSpec((_RB, n), lambda i: (i, 0))
    hid_blk_b = pl.BlockSpec((_RB, nhid), lambda i: (i, 0))
    one_blk_b = pl.BlockSpec((_RB, 1), lambda i: (i, 0))
    scalar = pl.BlockSpec((1, 1), lambda i: (0, 0))

    f8 = jnp.float8_e4m3fn
    h0, h0b, den, adjb, adjf8 = pl.pallas_call(
        _pass_a_kernel,
        grid=(nblk_a,),
        in_specs=[scalar, scalar, row_blk_a,
                  pl.BlockSpec((n, nhid), lambda i: (0, 0)),
                  pl.BlockSpec((1, nhid), lambda i: (0, 0))],
        out_specs=[hid_blk_a, hid_blk_a, one_blk_a, row_blk_a, row_blk_a],
        out_shape=[jax.ShapeDtypeStruct((n, nhid), f32),
                   jax.ShapeDtypeStruct((n, nhid), bf16),
                   jax.ShapeDtypeStruct((n, 1), f32),
                   jax.ShapeDtypeStruct((n, n), bf16),
                   jax.ShapeDtypeStruct((n, n), f8)],
        compiler_params=params,
    )(alpha, beta, adj, xwb, b1r)

    crf_call = pl.pallas_call(
        _crf_kernel,
        grid=(nblk_b,),
        in_specs=[scalar, scalar, row_blk_b,
                  pl.BlockSpec((n, nhid), lambda i: (0, 0)),
                  hid_blk_b, one_blk_b],
        out_specs=hid_blk_b,
        out_shape=jax.ShapeDtypeStruct((n, nhid), bf16),
        compiler_params=params,
    )
    ht = h0b
    for _ in range(2):
        ht = crf_call(alpha, beta, adjf8, ht, h0, den)

    qb = pl.pallas_call(
        _crf_ln_kernel,
        grid=(nblk_b,),
        in_specs=[scalar, scalar, row_blk_b,
                  pl.BlockSpec((n, nhid), lambda i: (0, 0)),
                  hid_blk_b, one_blk_b,
                  pl.BlockSpec((1, nhid), lambda i: (0, 0)),
                  pl.BlockSpec((1, nhid), lambda i: (0, 0)),
                  pl.BlockSpec((nhid, ncls), lambda i: (0, 0))],
        out_specs=pl.BlockSpec((_RB, ncls), lambda i: (i, 0)),
        out_shape=jax.ShapeDtypeStruct((n, ncls), bf16),
        compiler_params=params,
    )(alpha, beta, adjf8, ht, h0, den, gr, lbr, w2)

    out = pl.pallas_call(
        _gc2_kernel,
        grid=(nblk_b,),
        in_specs=[row_blk_b,
                  pl.BlockSpec((n, ncls), lambda i: (0, 0)),
                  pl.BlockSpec((1, ncls), lambda i: (0, 0))],
        out_specs=pl.BlockSpec((_RB, ncls), lambda i: (i, 0)),
        out_shape=jax.ShapeDtypeStruct((n, ncls), f32),
        compiler_params=params,
    )(adjb, qb, b2r)
    return out
